# baseline (speedup 1.0000x reference)
import functools

import jax
import jax.numpy as jnp
from jax import lax
from jax.experimental import pallas as pl
from jax.experimental.pallas import tpu as pltpu
from jax.experimental.pallas import tpu_sc as plsc

_N_EMB = 4_000_000
_DIM = 5
_B = 16384
_F = 26
_TOT = _B * _F
_NW = 32
_PER_W = _TOT // _NW
_FOLD_C = 524288
_FOLD_GRID = -(-_N_EMB // _FOLD_C)


def _fold_body(wb_ref, tT_ref, out_ref):
    acc = jnp.full((_FOLD_C,), wb_ref[_DIM], jnp.float32)
    for d in range(_DIM):
        acc = acc + tT_ref[d, :] * wb_ref[d]
    out_ref[...] = acc


_fold = pl.pallas_call(
    _fold_body,
    grid=(_FOLD_GRID,),
    in_specs=[
        pl.BlockSpec(memory_space=pltpu.SMEM),
        pl.BlockSpec((_DIM, _FOLD_C), lambda i: (0, i)),
    ],
    out_specs=pl.BlockSpec((_FOLD_C,), lambda i: (i,)),
    out_shape=jax.ShapeDtypeStruct((_N_EMB,), jnp.float32),
)

_mesh = plsc.VectorSubcoreMesh(core_axis_name="c", subcore_axis_name="s")


@functools.partial(
    pl.kernel,
    out_type=jax.ShapeDtypeStruct((_TOT,), jnp.float32),
    mesh=_mesh,
    scratch_types=[
        pltpu.VMEM((_PER_W,), jnp.int32),
        pltpu.VMEM((_PER_W,), jnp.float32),
        pltpu.SemaphoreType.DMA,
    ],
)
def _sc_gather(idx_hbm, folded_hbm, out_hbm, idx_v, val_v, sem):
    wid = lax.axis_index("s") * 2 + lax.axis_index("c")
    base = wid * _PER_W
    pltpu.sync_copy(idx_hbm.at[pl.ds(base, _PER_W)], idx_v)
    pltpu.async_copy(folded_hbm.at[idx_v], val_v, sem).wait()
    pltpu.sync_copy(val_v, out_hbm.at[pl.ds(base, _PER_W)])


def kernel(input, table, W, b):
    wb = jnp.concatenate(
        [W.reshape(_DIM), b.reshape(1), jnp.zeros((2,), jnp.float32)]
    ).astype(jnp.float32)
    folded = _fold(wb, table.T)
    idx = input.T.reshape(_TOT).astype(jnp.int32)
    out = _sc_gather(idx, folded)
    return out.reshape(_F, _B).T.reshape(_B, _F, 1)


# speedup vs baseline: 1.0691x; 1.0691x over previous
import functools

import jax
import jax.numpy as jnp
from jax import lax
from jax.experimental import pallas as pl
from jax.experimental.pallas import tpu as pltpu
from jax.experimental.pallas import tpu_sc as plsc

_N_EMB = 4_000_000
_DIM = 5
_B = 16384
_F = 26
_TOT = _B * _F
_NW = 32
_PER_W = _TOT // _NW
_FOLD_C = 262144
_FOLD_GRID = -(-_N_EMB // _FOLD_C)


def _fold_body(wb_ref, tT_ref, out_ref):
    acc = jnp.full((_FOLD_C,), wb_ref[_DIM], jnp.float32)
    for d in range(_DIM):
        acc = acc + tT_ref[d, :] * wb_ref[d]
    out_ref[...] = acc


_fold = pl.pallas_call(
    _fold_body,
    grid=(_FOLD_GRID,),
    in_specs=[
        pl.BlockSpec(memory_space=pltpu.SMEM),
        pl.BlockSpec((_DIM, _FOLD_C), lambda i: (0, i)),
    ],
    out_specs=pl.BlockSpec((_FOLD_C,), lambda i: (i,)),
    out_shape=jax.ShapeDtypeStruct((_N_EMB,), jnp.float32),
)

_mesh = plsc.VectorSubcoreMesh(core_axis_name="c", subcore_axis_name="s")


@functools.partial(
    pl.kernel,
    out_type=jax.ShapeDtypeStruct((_TOT,), jnp.float32),
    mesh=_mesh,
    scratch_types=[
        pltpu.VMEM((_PER_W,), jnp.int32),
        pltpu.VMEM((_PER_W,), jnp.float32),
        pltpu.SemaphoreType.DMA,
    ],
)
def _sc_gather(idx_hbm, folded_hbm, out_hbm, idx_v, val_v, sem):
    wid = lax.axis_index("s") * 2 + lax.axis_index("c")
    base = wid * _PER_W
    pltpu.sync_copy(idx_hbm.at[pl.ds(base, _PER_W)], idx_v)
    pltpu.async_copy(folded_hbm.at[idx_v], val_v, sem).wait()
    pltpu.sync_copy(val_v, out_hbm.at[pl.ds(base, _PER_W)])


def kernel(input, table, W, b):
    wb = jnp.concatenate(
        [W.reshape(_DIM), b.reshape(1), jnp.zeros((2,), jnp.float32)]
    ).astype(jnp.float32)
    folded = _fold(wb, table.T)
    idx = input.T.reshape(_TOT).astype(jnp.int32)
    out = _sc_gather(idx, folded)
    return jnp.transpose(out.reshape(_F, _B, 1), (1, 0, 2))


# speedup vs baseline: 1.0756x; 1.0061x over previous
import functools

import jax
import jax.numpy as jnp
from jax import lax
from jax.experimental import pallas as pl
from jax.experimental.pallas import tpu as pltpu
from jax.experimental.pallas import tpu_sc as plsc

_N_EMB = 4_000_000
_DIM = 5
_B = 16384
_F = 26
_TOT = _B * _F
_NW = 32
_PER_W = _TOT // _NW
_FOLD_C = 262144
_FOLD_GRID = -(-_N_EMB // _FOLD_C)


def _fold_body(w_ref, b_ref, tT_ref, out_ref):
    acc = jnp.full((_FOLD_C,), b_ref[0], jnp.float32)
    for d in range(_DIM):
        acc = acc + tT_ref[d, :] * w_ref[0, d]
    out_ref[...] = acc


_fold = pl.pallas_call(
    _fold_body,
    grid=(_FOLD_GRID,),
    in_specs=[
        pl.BlockSpec(memory_space=pltpu.SMEM),
        pl.BlockSpec(memory_space=pltpu.SMEM),
        pl.BlockSpec((_DIM, _FOLD_C), lambda i: (0, i)),
    ],
    out_specs=pl.BlockSpec((_FOLD_C,), lambda i: (i,)),
    out_shape=jax.ShapeDtypeStruct((_N_EMB,), jnp.float32),
)

_mesh = plsc.VectorSubcoreMesh(core_axis_name="c", subcore_axis_name="s")


@functools.partial(
    pl.kernel,
    out_type=jax.ShapeDtypeStruct((_TOT,), jnp.float32),
    mesh=_mesh,
    scratch_types=[
        pltpu.VMEM((_PER_W,), jnp.int32),
        pltpu.VMEM((_PER_W,), jnp.float32),
        pltpu.SemaphoreType.DMA,
    ],
)
def _sc_gather(idx_hbm, folded_hbm, out_hbm, idx_v, val_v, sem):
    wid = lax.axis_index("s") * 2 + lax.axis_index("c")
    base = wid * _PER_W
    pltpu.sync_copy(idx_hbm.at[pl.ds(base, _PER_W)], idx_v)
    pltpu.async_copy(folded_hbm.at[idx_v], val_v, sem).wait()
    pltpu.sync_copy(val_v, out_hbm.at[pl.ds(base, _PER_W)])


def kernel(input, table, W, b):
    folded = _fold(W, b, table.T)
    idx = input.T.reshape(_TOT).astype(jnp.int32)
    out = _sc_gather(idx, folded)
    return jnp.transpose(out.reshape(_F, _B, 1), (1, 0, 2))
